# Initial kernel scaffold; baseline (speedup 1.0000x reference)
import functools

import jax
import jax.numpy as jnp
from jax import lax
from jax.experimental import pallas as pl
from jax.experimental.pallas import tpu as pltpu
from jax.experimental.pallas import tpu_sc as plsc

N = 10000
E = 320000
NB = 2 * N
D = 64
NC, NS = 2, 16
NW = NC * NS
CH = 128
TRASH = NB
ACC_ROWS = 20480
RZ = ACC_ROWS // NS
EA = 2 * E
NCHUNK = ((EA + NW * CH - 1) // (NW * CH)) * NW
CPW = NCHUNK // NW
EPAD = NCHUNK * CH

_mesh = plsc.VectorSubcoreMesh(core_axis_name="c", subcore_axis_name="s")


@functools.partial(
    pl.kernel,
    out_type=jax.ShapeDtypeStruct((NC, ACC_ROWS, D), jnp.float32),
    mesh=_mesh,
    scratch_types=[
        pltpu.VMEM((CH,), jnp.int32),
        pltpu.VMEM((CH,), jnp.int32),
        pltpu.VMEM((CH, D), jnp.float32),
        pltpu.VMEM_SHARED((ACC_ROWS, D), jnp.float32),
        pltpu.SemaphoreType.DMA,
    ],
)
def _sc_agg(table_h, src_h, dst_h, zeros_h, out_h, s_idx, d_idx, rows, acc, sem):
    c = lax.axis_index("c")
    s = lax.axis_index("s")
    wid = c * NS + s
    pltpu.sync_copy(zeros_h.at[pl.ds(s * RZ, RZ)], acc.at[pl.ds(s * RZ, RZ)])
    plsc.subcore_barrier()

    def body(k, carry):
        chunk = wid * CPW + k
        pltpu.sync_copy(src_h.at[chunk], s_idx)
        pltpu.sync_copy(dst_h.at[chunk], d_idx)
        pltpu.async_copy(table_h.at[s_idx], rows, sem).wait()
        pltpu.sync_copy(rows, acc.at[d_idx], add=True)
        return carry

    lax.fori_loop(0, CPW, body, 0)
    plsc.subcore_barrier()
    pltpu.sync_copy(acc.at[pl.ds(s * RZ, RZ)], out_h.at[c].at[pl.ds(s * RZ, RZ)])


@functools.partial(
    pl.kernel,
    out_type=jax.ShapeDtypeStruct((NC, ACC_ROWS, 16), jnp.float32),
    mesh=_mesh,
    scratch_types=[
        pltpu.VMEM((CH,), jnp.int32),
        pltpu.VMEM((CH, 16), jnp.float32),
        pltpu.VMEM_SHARED((ACC_ROWS, 16), jnp.float32),
    ],
)
def _sc_deg(dst_h, zeros_h, ones_h, out_h, d_idx, ones_v, acc):
    c = lax.axis_index("c")
    s = lax.axis_index("s")
    wid = c * NS + s
    pltpu.sync_copy(zeros_h.at[pl.ds(s * RZ, RZ)], acc.at[pl.ds(s * RZ, RZ)])
    pltpu.sync_copy(ones_h, ones_v)
    plsc.subcore_barrier()

    def body(k, carry):
        chunk = wid * CPW + k
        pltpu.sync_copy(dst_h.at[chunk], d_idx)
        pltpu.sync_copy(ones_v, acc.at[d_idx], add=True)
        return carry

    lax.fori_loop(0, CPW, body, 0)
    plsc.subcore_barrier()
    pltpu.sync_copy(acc.at[pl.ds(s * RZ, RZ)], out_h.at[c].at[pl.ds(s * RZ, RZ)])


_BR = 1000
_G = NB // _BR


def _dinv_body(p0_ref, p1_ref, o_ref):
    deg = p0_ref[0, :, 0:1] + p1_ref[0, :, 0:1] + 1.0
    o_ref[...] = jnp.broadcast_to(lax.rsqrt(deg), o_ref.shape)


def _dinv(parts16):
    return pl.pallas_call(
        _dinv_body,
        grid=(_G,),
        in_specs=[
            pl.BlockSpec((1, _BR, 16), lambda i: (0, i, 0)),
            pl.BlockSpec((1, _BR, 16), lambda i: (1, i, 0)),
        ],
        out_specs=pl.BlockSpec((_BR, D), lambda i: (i, 0)),
        out_shape=jax.ShapeDtypeStruct((NB, D), jnp.float32),
    )(parts16, parts16)


def _mm1_body(x_ref, w_ref, dinv_ref, o_ref):
    h = jnp.dot(x_ref[...], w_ref[0], preferred_element_type=jnp.float32)
    o_ref[...] = h * dinv_ref[...]


def _mm1(x, w, dinv):
    fin = x.shape[1]
    return pl.pallas_call(
        _mm1_body,
        grid=(_G,),
        in_specs=[
            pl.BlockSpec((_BR, fin), lambda i: (i, 0)),
            pl.BlockSpec((1, fin, D), lambda i: (i // (_G // 2), 0, 0)),
            pl.BlockSpec((_BR, D), lambda i: (i, 0)),
        ],
        out_specs=pl.BlockSpec((_BR, D), lambda i: (i, 0)),
        out_shape=jax.ShapeDtypeStruct((NB, D), jnp.float32),
    )(x, w, dinv)


def _mid_body(p0_ref, p1_ref, hs_ref, dinv_ref, b_ref, w_ref, o_ref):
    b = b_ref[0, 0:1, :]
    x = dinv_ref[...] * (p0_ref[0] + p1_ref[0] + hs_ref[...]) + b
    x = jnp.maximum(x, 0.0)
    h = jnp.dot(x, w_ref[0], preferred_element_type=jnp.float32)
    o_ref[...] = h * dinv_ref[...]


def _mid(parts, hs, dinv, b, w):
    return pl.pallas_call(
        _mid_body,
        grid=(_G,),
        in_specs=[
            pl.BlockSpec((1, _BR, D), lambda i: (0, i, 0)),
            pl.BlockSpec((1, _BR, D), lambda i: (1, i, 0)),
            pl.BlockSpec((_BR, D), lambda i: (i, 0)),
            pl.BlockSpec((_BR, D), lambda i: (i, 0)),
            pl.BlockSpec((1, 8, D), lambda i: (i // (_G // 2), 0, 0)),
            pl.BlockSpec((1, D, D), lambda i: (i // (_G // 2), 0, 0)),
        ],
        out_specs=pl.BlockSpec((_BR, D), lambda i: (i, 0)),
        out_shape=jax.ShapeDtypeStruct((NB, D), jnp.float32),
    )(parts, parts, hs, dinv, b, w)


def _fin_body(p0_ref, p1_ref, hs_ref, dinv_ref, b_ref, o_ref):
    b = b_ref[0, 0:1, :]
    o_ref[...] = dinv_ref[...] * (p0_ref[0] + p1_ref[0] + hs_ref[...]) + b


def _fin(parts, hs, dinv, b):
    return pl.pallas_call(
        _fin_body,
        grid=(_G,),
        in_specs=[
            pl.BlockSpec((1, _BR, D), lambda i: (0, i, 0)),
            pl.BlockSpec((1, _BR, D), lambda i: (1, i, 0)),
            pl.BlockSpec((_BR, D), lambda i: (i, 0)),
            pl.BlockSpec((_BR, D), lambda i: (i, 0)),
            pl.BlockSpec((1, 8, D), lambda i: (i // (_G // 2), 0, 0)),
        ],
        out_specs=pl.BlockSpec((_BR, D), lambda i: (i, 0)),
        out_shape=jax.ShapeDtypeStruct((NB, D), jnp.float32),
    )(parts, parts, hs, dinv, b)


def kernel(x_data_matrix, x_edge_index, y_data_matrix, y_edge_index,
           Wx1, bx1, Wx2, bx2, Wx3, bx3, Wx4, bx4, Wx5, bx5,
           Wy1, by1, Wy2, by2, Wy3, by3, Wy4, by4, Wy5, by5):
    pad = EPAD - EA
    srcs = jnp.concatenate([
        x_edge_index[0], y_edge_index[0] + N,
        jnp.zeros((pad,), jnp.int32)])
    dsts = jnp.concatenate([
        x_edge_index[1], y_edge_index[1] + N,
        jnp.full((pad,), TRASH, jnp.int32)])
    src2d = srcs.reshape(NCHUNK, CH)
    dst2d = dsts.reshape(NCHUNK, CH)

    zeros64 = jnp.zeros((ACC_ROWS, D), jnp.float32)
    zeros16 = jnp.zeros((ACC_ROWS, 16), jnp.float32)
    ones = jnp.ones((CH, 16), jnp.float32)

    ws = [(jnp.stack([Wx1, Wy1]), jnp.stack([bx1, by1])),
          (jnp.stack([Wx2, Wy2]), jnp.stack([bx2, by2])),
          (jnp.stack([Wx3, Wy3]), jnp.stack([bx3, by3])),
          (jnp.stack([Wx4, Wy4]), jnp.stack([bx4, by4])),
          (jnp.stack([Wx5, Wy5]), jnp.stack([bx5, by5]))]
    bs = [jnp.broadcast_to(b[:, None, :], (2, 8, D)) for _, b in ws]

    xall = jnp.concatenate([x_data_matrix, y_data_matrix], axis=0)

    degp = _sc_deg(dst2d, zeros16, ones)
    dinv = _dinv(degp)

    hs = _mm1(xall, ws[0][0], dinv)
    out = None
    for li in range(5):
        parts = _sc_agg(hs, src2d, dst2d, zeros64)
        if li < 4:
            hs = _mid(parts, hs, dinv, bs[li], ws[li + 1][0])
        else:
            out = _fin(parts, hs, dinv, bs[4])
    return out[:N], out[N:]


# speedup vs baseline: 9.4619x; 9.4619x over previous
import functools

import jax
import jax.numpy as jnp
from jax import lax
from jax.experimental import pallas as pl
from jax.experimental.pallas import tpu as pltpu
from jax.experimental.pallas import tpu_sc as plsc

N = 10000
E = 320000
NB = 2 * N
D = 64
NC, NS = 2, 16
NW = NC * NS
CH = 128
TRASH = NB
ACC_ROWS = 20480
RZ = ACC_ROWS // NS
EA = 2 * E
NCHUNK = ((EA + NW * CH - 1) // (NW * CH)) * NW
CPW = NCHUNK // NW
EPAD = NCHUNK * CH

_mesh = plsc.VectorSubcoreMesh(core_axis_name="c", subcore_axis_name="s")


@functools.partial(
    pl.kernel,
    out_type=jax.ShapeDtypeStruct((NC, ACC_ROWS, D), jnp.float32),
    mesh=_mesh,
    scratch_types=[
        pltpu.VMEM((CH,), jnp.int32),
        pltpu.VMEM((CH,), jnp.int32),
        pltpu.VMEM((CH, D), jnp.float32),
        pltpu.VMEM_SHARED((ACC_ROWS, D), jnp.float32),
        pltpu.SemaphoreType.DMA,
    ],
    compiler_params=pltpu.CompilerParams(use_tc_tiling_on_sc=False),
)
def _sc_agg(table_h, src_h, dst_h, zeros_h, out_h, s_idx, d_idx, rows, acc, sem):
    c = lax.axis_index("c")
    s = lax.axis_index("s")
    wid = c * NS + s
    pltpu.sync_copy(zeros_h.at[pl.ds(s * RZ, RZ)], acc.at[pl.ds(s * RZ, RZ)])
    plsc.subcore_barrier()

    def body(k, carry):
        chunk = wid * CPW + k
        pltpu.sync_copy(src_h.at[chunk], s_idx)
        pltpu.sync_copy(dst_h.at[chunk], d_idx)
        pltpu.async_copy(table_h.at[s_idx], rows, sem).wait()
        pltpu.sync_copy(rows, acc.at[d_idx], add=True)
        return carry

    lax.fori_loop(0, CPW, body, 0)
    plsc.subcore_barrier()
    pltpu.sync_copy(acc.at[pl.ds(s * RZ, RZ)], out_h.at[c].at[pl.ds(s * RZ, RZ)])


@functools.partial(
    pl.kernel,
    out_type=jax.ShapeDtypeStruct((NC, ACC_ROWS, 16), jnp.float32),
    mesh=_mesh,
    scratch_types=[
        pltpu.VMEM((CH,), jnp.int32),
        pltpu.VMEM((CH, 16), jnp.float32),
        pltpu.VMEM_SHARED((ACC_ROWS, 16), jnp.float32),
    ],
    compiler_params=pltpu.CompilerParams(use_tc_tiling_on_sc=False),
)
def _sc_deg(dst_h, zeros_h, ones_h, out_h, d_idx, ones_v, acc):
    c = lax.axis_index("c")
    s = lax.axis_index("s")
    wid = c * NS + s
    pltpu.sync_copy(zeros_h.at[pl.ds(s * RZ, RZ)], acc.at[pl.ds(s * RZ, RZ)])
    pltpu.sync_copy(ones_h, ones_v)
    plsc.subcore_barrier()

    def body(k, carry):
        chunk = wid * CPW + k
        pltpu.sync_copy(dst_h.at[chunk], d_idx)
        pltpu.sync_copy(ones_v, acc.at[d_idx], add=True)
        return carry

    lax.fori_loop(0, CPW, body, 0)
    plsc.subcore_barrier()
    pltpu.sync_copy(acc.at[pl.ds(s * RZ, RZ)], out_h.at[c].at[pl.ds(s * RZ, RZ)])


_BR = 1000
_G = NB // _BR


def _dinv_body(p0_ref, p1_ref, o_ref):
    deg = p0_ref[0, :, 0:1] + p1_ref[0, :, 0:1] + 1.0
    o_ref[...] = jnp.broadcast_to(lax.rsqrt(deg), o_ref.shape)


def _dinv(parts16):
    return pl.pallas_call(
        _dinv_body,
        grid=(_G,),
        in_specs=[
            pl.BlockSpec((1, _BR, 16), lambda i: (0, i, 0)),
            pl.BlockSpec((1, _BR, 16), lambda i: (1, i, 0)),
        ],
        out_specs=pl.BlockSpec((_BR, D), lambda i: (i, 0)),
        out_shape=jax.ShapeDtypeStruct((NB, D), jnp.float32),
    )(parts16, parts16)


def _mm1_body(x_ref, w_ref, dinv_ref, o_ref):
    h = jnp.dot(x_ref[...], w_ref[0], preferred_element_type=jnp.float32)
    o_ref[...] = h * dinv_ref[...]


def _mm1(x, w, dinv):
    fin = x.shape[1]
    return pl.pallas_call(
        _mm1_body,
        grid=(_G,),
        in_specs=[
            pl.BlockSpec((_BR, fin), lambda i: (i, 0)),
            pl.BlockSpec((1, fin, D), lambda i: (i // (_G // 2), 0, 0)),
            pl.BlockSpec((_BR, D), lambda i: (i, 0)),
        ],
        out_specs=pl.BlockSpec((_BR, D), lambda i: (i, 0)),
        out_shape=jax.ShapeDtypeStruct((NB, D), jnp.float32),
    )(x, w, dinv)


def _mid_body(p0_ref, p1_ref, hs_ref, dinv_ref, b_ref, w_ref, o_ref):
    b = b_ref[0, 0:1, :]
    x = dinv_ref[...] * (p0_ref[0] + p1_ref[0] + hs_ref[...]) + b
    x = jnp.maximum(x, 0.0)
    h = jnp.dot(x, w_ref[0], preferred_element_type=jnp.float32)
    o_ref[...] = h * dinv_ref[...]


def _mid(parts, hs, dinv, b, w):
    return pl.pallas_call(
        _mid_body,
        grid=(_G,),
        in_specs=[
            pl.BlockSpec((1, _BR, D), lambda i: (0, i, 0)),
            pl.BlockSpec((1, _BR, D), lambda i: (1, i, 0)),
            pl.BlockSpec((_BR, D), lambda i: (i, 0)),
            pl.BlockSpec((_BR, D), lambda i: (i, 0)),
            pl.BlockSpec((1, 8, D), lambda i: (i // (_G // 2), 0, 0)),
            pl.BlockSpec((1, D, D), lambda i: (i // (_G // 2), 0, 0)),
        ],
        out_specs=pl.BlockSpec((_BR, D), lambda i: (i, 0)),
        out_shape=jax.ShapeDtypeStruct((NB, D), jnp.float32),
    )(parts, parts, hs, dinv, b, w)


def _fin_body(p0_ref, p1_ref, hs_ref, dinv_ref, b_ref, o_ref):
    b = b_ref[0, 0:1, :]
    o_ref[...] = dinv_ref[...] * (p0_ref[0] + p1_ref[0] + hs_ref[...]) + b


def _fin(parts, hs, dinv, b):
    return pl.pallas_call(
        _fin_body,
        grid=(_G,),
        in_specs=[
            pl.BlockSpec((1, _BR, D), lambda i: (0, i, 0)),
            pl.BlockSpec((1, _BR, D), lambda i: (1, i, 0)),
            pl.BlockSpec((_BR, D), lambda i: (i, 0)),
            pl.BlockSpec((_BR, D), lambda i: (i, 0)),
            pl.BlockSpec((1, 8, D), lambda i: (i // (_G // 2), 0, 0)),
        ],
        out_specs=pl.BlockSpec((_BR, D), lambda i: (i, 0)),
        out_shape=jax.ShapeDtypeStruct((NB, D), jnp.float32),
    )(parts, parts, hs, dinv, b)


def kernel(x_data_matrix, x_edge_index, y_data_matrix, y_edge_index,
           Wx1, bx1, Wx2, bx2, Wx3, bx3, Wx4, bx4, Wx5, bx5,
           Wy1, by1, Wy2, by2, Wy3, by3, Wy4, by4, Wy5, by5):
    pad = EPAD - EA
    srcs = jnp.concatenate([
        x_edge_index[0], y_edge_index[0] + N,
        jnp.zeros((pad,), jnp.int32)])
    dsts = jnp.concatenate([
        x_edge_index[1], y_edge_index[1] + N,
        jnp.full((pad,), TRASH, jnp.int32)])
    src2d = srcs.reshape(NCHUNK, CH)
    dst2d = dsts.reshape(NCHUNK, CH)

    zeros64 = jnp.zeros((ACC_ROWS, D), jnp.float32)
    zeros16 = jnp.zeros((ACC_ROWS, 16), jnp.float32)
    ones = jnp.ones((CH, 16), jnp.float32)

    ws = [(jnp.stack([Wx1, Wy1]), jnp.stack([bx1, by1])),
          (jnp.stack([Wx2, Wy2]), jnp.stack([bx2, by2])),
          (jnp.stack([Wx3, Wy3]), jnp.stack([bx3, by3])),
          (jnp.stack([Wx4, Wy4]), jnp.stack([bx4, by4])),
          (jnp.stack([Wx5, Wy5]), jnp.stack([bx5, by5]))]
    bs = [jnp.broadcast_to(b[:, None, :], (2, 8, D)) for _, b in ws]

    xall = jnp.concatenate([x_data_matrix, y_data_matrix], axis=0)

    degp = _sc_deg(dst2d, zeros16, ones)
    dinv = _dinv(degp)

    hs = _mm1(xall, ws[0][0], dinv)
    out = None
    for li in range(5):
        parts = _sc_agg(hs, src2d, dst2d, zeros64)
        if li < 4:
            hs = _mid(parts, hs, dinv, bs[li], ws[li + 1][0])
        else:
            out = _fin(parts, hs, dinv, bs[4])
    return out[:N], out[N:]
